# Initial kernel scaffold; baseline (speedup 1.0000x reference)
import jax
import jax.numpy as jnp
from jax.experimental import pallas as pl


def kernel(x, Pi, centroids, boundaries):
    raise NotImplementedError("write your pallas kernel here")


# speedup vs baseline: 2.7082x; 2.7082x over previous
import jax
import jax.numpy as jnp
from jax.experimental import pallas as pl
from jax.experimental.pallas import tpu as pltpu

BM = 512
BN = 512


def _quant_matmul_kernel(b_ref, c_ref, x_ref, pi_ref, idx_ref, yhat_ref):
    y = jnp.dot(x_ref[...], pi_ref[...], preferred_element_type=jnp.float32)
    idx = jnp.zeros(y.shape, jnp.int32)
    yhat = jnp.full(y.shape, c_ref[0], jnp.float32)
    for k in range(b_ref.shape[0]):
        m = y > b_ref[k]
        idx = idx + m.astype(jnp.int32)
        yhat = yhat + jnp.where(m, c_ref[k + 1] - c_ref[k], 0.0)
    idx_ref[...] = idx
    yhat_ref[...] = yhat.astype(jnp.bfloat16)


def _unrotate_kernel(yhat_ref, pi_ref, out_ref):
    out_ref[...] = jax.lax.dot_general(
        yhat_ref[...], pi_ref[...],
        dimension_numbers=(((1,), (1,)), ((), ())),
        preferred_element_type=jnp.float32,
    )


def kernel(x, Pi, centroids, boundaries):
    M, d = x.shape
    grid = (M // BM, d // BN)

    idx, yhat = pl.pallas_call(
        _quant_matmul_kernel,
        grid=grid,
        in_specs=[
            pl.BlockSpec(memory_space=pltpu.SMEM),
            pl.BlockSpec(memory_space=pltpu.SMEM),
            pl.BlockSpec((BM, d), lambda i, j: (i, 0)),
            pl.BlockSpec((d, BN), lambda i, j: (0, j)),
        ],
        out_specs=[
            pl.BlockSpec((BM, BN), lambda i, j: (i, j)),
            pl.BlockSpec((BM, BN), lambda i, j: (i, j)),
        ],
        out_shape=[
            jax.ShapeDtypeStruct((M, d), jnp.int32),
            jax.ShapeDtypeStruct((M, d), jnp.bfloat16),
        ],
        compiler_params=pltpu.CompilerParams(
            dimension_semantics=("parallel", "arbitrary"),
        ),
    )(boundaries, centroids, x, Pi)

    pi_bf = Pi.astype(jnp.bfloat16)
    x_hat = pl.pallas_call(
        _unrotate_kernel,
        grid=grid,
        in_specs=[
            pl.BlockSpec((BM, d), lambda i, j: (i, 0)),
            pl.BlockSpec((BN, d), lambda i, j: (j, 0)),
        ],
        out_specs=pl.BlockSpec((BM, BN), lambda i, j: (i, j)),
        out_shape=jax.ShapeDtypeStruct((M, d), jnp.float32),
        compiler_params=pltpu.CompilerParams(
            dimension_semantics=("parallel", "arbitrary"),
        ),
    )(yhat, pi_bf)

    return (x_hat, idx)


# speedup vs baseline: 2.9054x; 1.0728x over previous
import jax
import jax.numpy as jnp
from jax.experimental import pallas as pl
from jax.experimental.pallas import tpu as pltpu

BM = 512
BN = 512


def _quant_matmul_kernel(b_ref, c_ref, x_ref, pi_ref, idx_ref, yhat_ref):
    y = jnp.dot(x_ref[...], pi_ref[...], preferred_element_type=jnp.float32)
    u = jnp.zeros(y.shape, jnp.float32)
    for k in range(b_ref.shape[0]):
        m = y > b_ref[k]
        u = u + jnp.where(m, (c_ref[k + 1] - c_ref[k]) + 16.0, 0.0)
    idx_f = jnp.floor(u * (1.0 / 16.0))
    idx_ref[...] = idx_f.astype(jnp.int32)
    yhat_ref[...] = ((u - 16.0 * idx_f) + c_ref[0]).astype(jnp.bfloat16)


def _unrotate_kernel(yhat_ref, pi_ref, out_ref):
    out_ref[...] = jax.lax.dot_general(
        yhat_ref[...], pi_ref[...],
        dimension_numbers=(((1,), (1,)), ((), ())),
        preferred_element_type=jnp.float32,
    )


def kernel(x, Pi, centroids, boundaries):
    M, d = x.shape
    grid = (M // BM, d // BN)

    x_bf = x.astype(jnp.bfloat16)
    pi_bf = Pi.astype(jnp.bfloat16)

    idx, yhat = pl.pallas_call(
        _quant_matmul_kernel,
        grid=grid,
        in_specs=[
            pl.BlockSpec(memory_space=pltpu.SMEM),
            pl.BlockSpec(memory_space=pltpu.SMEM),
            pl.BlockSpec((BM, d), lambda i, j: (i, 0)),
            pl.BlockSpec((d, BN), lambda i, j: (0, j)),
        ],
        out_specs=[
            pl.BlockSpec((BM, BN), lambda i, j: (i, j)),
            pl.BlockSpec((BM, BN), lambda i, j: (i, j)),
        ],
        out_shape=[
            jax.ShapeDtypeStruct((M, d), jnp.int32),
            jax.ShapeDtypeStruct((M, d), jnp.bfloat16),
        ],
        compiler_params=pltpu.CompilerParams(
            dimension_semantics=("parallel", "arbitrary"),
        ),
    )(boundaries, centroids, x_bf, pi_bf)

    x_hat = pl.pallas_call(
        _unrotate_kernel,
        grid=grid,
        in_specs=[
            pl.BlockSpec((BM, d), lambda i, j: (i, 0)),
            pl.BlockSpec((BN, d), lambda i, j: (j, 0)),
        ],
        out_specs=pl.BlockSpec((BM, BN), lambda i, j: (i, j)),
        out_shape=jax.ShapeDtypeStruct((M, d), jnp.float32),
        compiler_params=pltpu.CompilerParams(
            dimension_semantics=("parallel", "arbitrary"),
        ),
    )(yhat, pi_bf)

    return (x_hat, idx)
